# Initial kernel scaffold; baseline (speedup 1.0000x reference)
import jax
import jax.numpy as jnp
from jax.experimental import pallas as pl


def kernel(x, edge_index, edge_attr, batch, ph_enc, temp_enc, box_idx, receptor_flag, in_proj_w, in_proj_b, L0_lin_l, L0_lin_r, L0_lin_edge, L0_att, L0_bias, L0_bn_w, L0_bn_b, L1_lin_l, L1_lin_r, L1_lin_edge, L1_att, L1_bias, L1_bn_w, L1_bn_b, L2_lin_l, L2_lin_r, L2_lin_edge, L2_att, L2_bias, L2_bn_w, L2_bn_b, L3_lin_l, L3_lin_r, L3_lin_edge, L3_att, L3_bias, L3_bn_w, L3_bn_b, L3_skip, post_w, post_b, box_table, cp_w1, cp_b1, cp_w2, cp_b2, mlp_w1, mlp_b1, mlp_w2, mlp_b2, mlp_w3, mlp_b3):
    raise NotImplementedError("write your pallas kernel here")


# speedup vs baseline: 8.7158x; 8.7158x over previous
import dataclasses
import functools

import jax
import jax.numpy as jnp
from jax import lax
from jax.experimental import pallas as pl
from jax.experimental.pallas import tpu as pltpu
from jax.experimental.pallas import tpu_sc as plsc

_N = 10000
_E = 640000
_HID = 128
_NH = 4
_NB = 256

_EP = 655360
_EB = 8192
_NEB = _EP // _EB
_NACC = 10240
_NC = 2
_NS = 16
_NW = _NC * _NS
_CH = 128
_IDXROWS = _EP // _CH
_ROWS_PW = _IDXROWS // _NW
_ROWS_PT = _NACC // _NS

def _vec_mesh():
    return plsc.VectorSubcoreMesh(core_axis_name="c", subcore_axis_name="s",
                                  num_cores=_NC)


def _sc_gather(xl, xr, src2d, dst2d):
    @functools.partial(
        pl.kernel,
        mesh=_vec_mesh(),
        out_type=[jax.ShapeDtypeStruct((_EP, _HID), jnp.float32),
                  jax.ShapeDtypeStruct((_EP, _HID), jnp.float32)],
        scratch_types=[
            pltpu.VMEM((16, _CH), jnp.int32),
            pltpu.VMEM((16, _CH), jnp.int32),
            pltpu.VMEM((_CH, _HID), jnp.float32),
            pltpu.VMEM((_CH, _HID), jnp.float32),
            pltpu.SemaphoreType.DMA,
            pltpu.SemaphoreType.DMA,
        ],
    )
    def k(xl_hbm, xr_hbm, src_hbm, dst_hbm, gl_hbm, gr_hbm,
          sidx, didx, rows_l, rows_r, sem_l, sem_r):
        wid = lax.axis_index("s") * _NC + lax.axis_index("c")
        row0 = wid * _ROWS_PW

        @pl.loop(0, _ROWS_PW // 16)
        def _(o):
            rb = row0 + o * 16
            pltpu.sync_copy(src_hbm.at[pl.ds(rb, 16), :], sidx)
            pltpu.sync_copy(dst_hbm.at[pl.ds(rb, 16), :], didx)

            @pl.loop(0, 16)
            def _(j):
                base = (rb + j) * _CH
                cl = pltpu.async_copy(xl_hbm.at[sidx.at[j]], rows_l, sem_l)
                cr = pltpu.async_copy(xr_hbm.at[didx.at[j]], rows_r, sem_r)
                cl.wait()
                cr.wait()
                pltpu.sync_copy(rows_l, gl_hbm.at[pl.ds(base, _CH), :])
                pltpu.sync_copy(rows_r, gr_hbm.at[pl.ds(base, _CH), :])

    return k(xl, xr, src2d, dst2d)


_ACCW = _NACC * 8
_EHALF = _EP // 2
_EQUARt = _EP // 4


def _sc_scatter(u0, u1, exs, dstp):
    cp = pltpu.CompilerParams()
    if "needs_layout_passes" in pltpu.CompilerParams.__dataclass_fields__:
        cp = dataclasses.replace(cp, needs_layout_passes=False)

    @functools.partial(
        pl.kernel,
        mesh=_vec_mesh(),
        compiler_params=cp,
        out_type=[jax.ShapeDtypeStruct((32 * _ACCW,), jnp.float32),
                  jax.ShapeDtypeStruct((16 * _NACC,), jnp.float32)],
        scratch_types=[
            pltpu.VMEM((_ACCW,), jnp.float32),
            pltpu.VMEM((_NACC,), jnp.float32),
            pltpu.VMEM((8, _CH), jnp.float32),
            pltpu.VMEM((_CH, 16), jnp.float32),
            pltpu.VMEM((_CH,), jnp.int32),
        ],
    )
    def k(u0_hbm, u1_hbm, exs_hbm, dst_hbm, outm_hbm, outd_hbm,
          acc, dacc, vb, eb, dv):
        c = lax.axis_index("c")
        s = lax.axis_index("s")
        strip = lax.rem(s, 8)
        half = s // 8
        zero16 = jnp.zeros((16,), jnp.float32)
        iota16 = lax.iota(jnp.int32, 16)
        rowpat = iota16 >> 3
        colpat = iota16 & 7

        @pl.loop(0, _ACCW // 16)
        def _(i):
            acc[pl.ds(i * 16, 16)] = zero16

        @pl.loop(0, _NACC // 16)
        def _(i):
            dacc[pl.ds(i * 16, 16)] = zero16

        ebase = half * _EHALF

        @pl.loop(0, _EHALF // _CH)
        def _(t):
            base = ebase + t * _CH
            pltpu.sync_copy(dst_hbm.at[pl.ds(base, _CH)], dv)

            @pl.when(c == 0)
            def _():
                pltpu.sync_copy(
                    u0_hbm.at[pl.ds(strip * 8, 8), pl.ds(base, _CH)], vb)

            @pl.when(c == 1)
            def _():
                pltpu.sync_copy(
                    u1_hbm.at[pl.ds(strip * 8, 8), pl.ds(base, _CH)], vb)

            @pl.loop(0, 64)
            def _(g):
                rows = g * 2 + rowpat
                dpair = plsc.load_gather(dv, [rows])
                idxv = dpair * 8 + colpat
                vals = plsc.load_gather(vb, [colpat, rows])
                plsc.addupdate_scatter(acc, [idxv], vals)

        @pl.when(c == 0)
        def _():
            d = lax.rem(s, 4)
            q = s // 4
            dsplat = jnp.zeros((16,), jnp.int32) + d

            @pl.loop(0, _EQUARt // _CH)
            def _(t):
                base = q * _EQUARt + t * _CH
                pltpu.sync_copy(dst_hbm.at[pl.ds(base, _CH)], dv)
                pltpu.sync_copy(exs_hbm.at[pl.ds(base, _CH), :], eb)

                @pl.loop(0, 8)
                def _(g):
                    rows = g * 16 + iota16
                    vals = plsc.load_gather(eb, [rows, dsplat])
                    idxv = dv[pl.ds(g * 16, 16)]
                    plsc.addupdate_scatter(dacc, [idxv], vals)

        wid = c * 16 + s
        pltpu.sync_copy(acc, outm_hbm.at[pl.ds(wid * _ACCW, _ACCW)])

        @pl.when(c == 0)
        def _():
            pltpu.sync_copy(dacc, outd_hbm.at[pl.ds(s * _NACC, _NACC)])

    outm, outd = k(u0, u1, exs, dstp)
    ms = outm.reshape(_NC, 2, 8, _NACC, 8).sum(axis=1)
    ms = jnp.transpose(ms, (2, 0, 1, 3)).reshape(_NACC, _HID)
    ds = outd.reshape(4, 4, _NACC).sum(axis=0).T
    return ms, ds


def _edge_body(gl_ref, gr_ref, ea_ref, le_ref, am_ref, xp_ref,
               u0_ref, u1_ref, exs_ref):
    i = pl.program_id(0)
    ee = jnp.dot(ea_ref[...], le_ref[...], preferred_element_type=jnp.float32)
    m = gl_ref[...] + gr_ref[...] + ee
    sm = jnp.where(m >= 0, m, 0.2 * m)
    a = jnp.dot(sm, am_ref[...], preferred_element_type=jnp.float32)
    a = jnp.minimum(a, 88.0)
    ex = jnp.exp(a)
    rowid = i * _EB + lax.broadcasted_iota(jnp.int32, (_EB, 1), 0)
    ex = ex * (rowid < _E).astype(jnp.float32)
    exe = jnp.dot(ex, xp_ref[...], preferred_element_type=jnp.float32)
    u = gl_ref[...] * exe
    u0_ref[...] = jnp.transpose(u[:, :64], (1, 0))
    u1_ref[...] = jnp.transpose(u[:, 64:], (1, 0))
    exs_ref[...] = jnp.concatenate([ex, jnp.zeros_like(ex)], axis=1)


def _edge_math(gl, gr, eap, lep, attmat, expand8):
    return pl.pallas_call(
        _edge_body,
        grid=(_NEB,),
        in_specs=[
            pl.BlockSpec((_EB, _HID), lambda i: (i, 0)),
            pl.BlockSpec((_EB, _HID), lambda i: (i, 0)),
            pl.BlockSpec((_EB, 16), lambda i: (i, 0)),
            pl.BlockSpec((16, _HID), lambda i: (0, 0)),
            pl.BlockSpec((_HID, 8), lambda i: (0, 0)),
            pl.BlockSpec((8, _HID), lambda i: (0, 0)),
        ],
        out_specs=[
            pl.BlockSpec((64, _EB), lambda i: (0, i)),
            pl.BlockSpec((64, _EB), lambda i: (0, i)),
            pl.BlockSpec((_EB, 16), lambda i: (i, 0)),
        ],
        out_shape=[jax.ShapeDtypeStruct((64, _EP), jnp.float32),
                   jax.ShapeDtypeStruct((64, _EP), jnp.float32),
                   jax.ShapeDtypeStruct((_EP, 16), jnp.float32)],
    )(gl, gr, eap, lep, attmat, expand8)


def _nk0_body(x_ref, w_ref, b_ref, ll_ref, lr_ref, x0_ref, xl_ref, xr_ref):
    x0 = jnp.dot(x_ref[...], w_ref[...],
                 preferred_element_type=jnp.float32) + b_ref[...]
    x0_ref[...] = x0
    xl_ref[...] = jnp.dot(x0, ll_ref[...], preferred_element_type=jnp.float32)
    xr_ref[...] = jnp.dot(x0, lr_ref[...], preferred_element_type=jnp.float32)


def _node0(x, w, b, ll, lr):
    return pl.pallas_call(
        _nk0_body,
        out_shape=[jax.ShapeDtypeStruct((_N, _HID), jnp.float32)] * 3,
    )(x, w, b, ll, lr)


def _combine(ms_ref, ds_ref, exp4_ref, bias_ref):
    msum = ms_ref[:_N, :]
    den = ds_ref[:_N, :]
    dfull = jnp.dot(den, exp4_ref[...], preferred_element_type=jnp.float32)
    return msum / (dfull + 1e-16) + bias_ref[...]


def _bn_elu(h, bnw_ref, bnb_ref):
    mu = jnp.mean(h, axis=0, keepdims=True)
    var = jnp.mean((h - mu) ** 2, axis=0, keepdims=True)
    h = (h - mu) / jnp.sqrt(var + 1e-5) * bnw_ref[...] + bnb_ref[...]
    return jnp.where(h > 0, h, jnp.exp(jnp.minimum(h, 0.0)) - 1.0)


def _nkmid_body(ms_ref, ds_ref, exp4_ref, bias_ref, bnw_ref, bnb_ref,
                xp_ref, ll_ref, lr_ref, x_ref, xl_ref, xr_ref):
    h = _combine(ms_ref, ds_ref, exp4_ref, bias_ref)
    h = _bn_elu(h, bnw_ref, bnb_ref)
    x = h + xp_ref[...]
    x_ref[...] = x
    xl_ref[...] = jnp.dot(x, ll_ref[...], preferred_element_type=jnp.float32)
    xr_ref[...] = jnp.dot(x, lr_ref[...], preferred_element_type=jnp.float32)


def _node_mid(ms, ds, exp4, bias, bnw, bnb, xprev, ll, lr):
    return pl.pallas_call(
        _nkmid_body,
        out_shape=[jax.ShapeDtypeStruct((_N, _HID), jnp.float32)] * 3,
    )(ms, ds, exp4, bias, bnw, bnb, xprev, ll, lr)


def _head_body(ms_ref, ds_ref, exp4_ref, hmean_ref, bias_ref, bnw_ref,
               bnb_ref, xp_ref, skip_ref, postw_ref, postb_ref, batch_ref,
               cont_ref, box_ref, boxt_ref, cw1_ref, cb1_ref, cw2_ref,
               cb2_ref, mw1_ref, mb1_ref, mw2_ref, mb2_ref, mw3_ref,
               mb3_ref, out_ref):
    msum = ms_ref[:_N, :]
    den = ds_ref[:_N, :]
    dfull = jnp.dot(den, exp4_ref[...], preferred_element_type=jnp.float32)
    hm = msum / (dfull + 1e-16)
    h = jnp.dot(hm, hmean_ref[...],
                preferred_element_type=jnp.float32) + bias_ref[...]
    h = _bn_elu(h, bnw_ref, bnb_ref)
    skip = jnp.dot(xp_ref[...], skip_ref[...],
                   preferred_element_type=jnp.float32)
    x4 = h + skip
    y = jnp.dot(x4, postw_ref[...],
                preferred_element_type=jnp.float32) + postb_ref[...]
    gid = lax.broadcasted_iota(jnp.int32, (_NB, _N), 0)
    onehot = (gid == batch_ref[...]).astype(jnp.float32)
    sums = jnp.dot(onehot, y, preferred_element_type=jnp.float32)
    cnt = jnp.sum(onehot, axis=1, keepdims=True)
    h_lig = sums / jnp.maximum(cnt, 1.0)
    bid = lax.broadcasted_iota(jnp.int32, (_NB, 8), 1)
    boh = (bid == box_ref[...]).astype(jnp.float32)
    bemb = jnp.dot(boh, boxt_ref[...], preferred_element_type=jnp.float32)
    cond = jnp.concatenate([cont_ref[...], bemb], axis=1)
    hc = jnp.dot(cond, cw1_ref[...],
                 preferred_element_type=jnp.float32) + cb1_ref[...]
    hc = jnp.maximum(hc, 0.0)
    hc = jnp.dot(hc, cw2_ref[...],
                 preferred_element_type=jnp.float32) + cb2_ref[...]
    hcat = jnp.concatenate([h_lig, hc], axis=1)
    h1 = jnp.maximum(jnp.dot(hcat, mw1_ref[...],
                             preferred_element_type=jnp.float32)
                     + mb1_ref[...], 0.0)
    h2 = jnp.maximum(jnp.dot(h1, mw2_ref[...],
                             preferred_element_type=jnp.float32)
                     + mb2_ref[...], 0.0)
    out_ref[...] = jnp.dot(h2, mw3_ref[...],
                           preferred_element_type=jnp.float32) + mb3_ref[...]


def _head(ms, ds, exp4, hmean, bias, bnw, bnb, xprev, skip, postw, postb,
          batch2d, cont, box2d, boxt, cw1, cb1, cw2, cb2,
          mw1, mb1, mw2, mb2, mw3, mb3):
    return pl.pallas_call(
        _head_body,
        out_shape=jax.ShapeDtypeStruct((_NB, 1), jnp.float32),
    )(ms, ds, exp4, hmean, bias, bnw, bnb, xprev, skip, postw, postb,
      batch2d, cont, box2d, boxt, cw1, cb1, cw2, cb2,
      mw1, mb1, mw2, mb2, mw3, mb3)


def kernel(x, edge_index, edge_attr, batch, ph_enc, temp_enc, box_idx,
           receptor_flag, in_proj_w, in_proj_b,
           L0_lin_l, L0_lin_r, L0_lin_edge, L0_att, L0_bias, L0_bn_w, L0_bn_b,
           L1_lin_l, L1_lin_r, L1_lin_edge, L1_att, L1_bias, L1_bn_w, L1_bn_b,
           L2_lin_l, L2_lin_r, L2_lin_edge, L2_att, L2_bias, L2_bn_w, L2_bn_b,
           L3_lin_l, L3_lin_r, L3_lin_edge, L3_att, L3_bias, L3_bn_w, L3_bn_b,
           L3_skip, post_w, post_b, box_table,
           cp_w1, cp_b1, cp_w2, cp_b2,
           mlp_w1, mlp_b1, mlp_w2, mlp_b2, mlp_w3, mlp_b3):
    f32 = jnp.float32
    pad = _EP - _E
    src2d = jnp.pad(edge_index[0].astype(jnp.int32), (0, pad)).reshape(
        _IDXROWS, _CH)
    dstp = jnp.pad(edge_index[1].astype(jnp.int32), (0, pad))
    dst2d = dstp.reshape(_IDXROWS, _CH)
    eap = jnp.pad(edge_attr, ((0, pad), (0, 3)))

    hidx = jnp.arange(_HID) // 32
    expand8 = (jnp.arange(8)[:, None] == hidx[None, :]).astype(f32)
    exp4 = (jnp.arange(4)[:, None] == hidx[None, :]).astype(f32)
    hmean = 0.25 * ((jnp.arange(_HID) % 32)[:, None]
                    == jnp.arange(32)[None, :]).astype(f32)

    layers = [
        (L0_lin_l, L0_lin_r, L0_lin_edge, L0_att, L0_bias, L0_bn_w, L0_bn_b),
        (L1_lin_l, L1_lin_r, L1_lin_edge, L1_att, L1_bias, L1_bn_w, L1_bn_b),
        (L2_lin_l, L2_lin_r, L2_lin_edge, L2_att, L2_bias, L2_bn_w, L2_bn_b),
        (L3_lin_l, L3_lin_r, L3_lin_edge, L3_att, L3_bias, L3_bn_w, L3_bn_b),
    ]

    def attmat_of(att):
        flat = att.reshape(_HID)
        return flat[:, None] * (hidx[:, None] == jnp.arange(8)[None, :]
                                ).astype(f32)

    xcur, xl, xr = _node0(x, in_proj_w, in_proj_b.reshape(1, _HID),
                          L0_lin_l, L0_lin_r)

    for i in range(4):
        ll, lr, le, att, bias, bnw, bnb = layers[i]
        lep = jnp.pad(le, ((0, 3), (0, 0)))
        gl, gr = _sc_gather(xl, xr, src2d, dst2d)
        u0, u1, exs = _edge_math(gl, gr, eap, lep, attmat_of(att), expand8)
        ms, ds = _sc_scatter(u0, u1, exs, dstp)
        if i < 3:
            nll, nlr = layers[i + 1][0], layers[i + 1][1]
            xcur, xl, xr = _node_mid(ms, ds, exp4, bias.reshape(1, _HID),
                                     bnw.reshape(1, _HID),
                                     bnb.reshape(1, _HID), xcur, nll, nlr)

    cont = jnp.stack([ph_enc, temp_enc, receptor_flag], axis=-1)
    bias3, bnw3, bnb3 = layers[3][4], layers[3][5], layers[3][6]
    return _head(ms, ds, exp4, hmean, bias3.reshape(1, 32),
                 bnw3.reshape(1, 32), bnb3.reshape(1, 32), xcur, L3_skip,
                 post_w, post_b.reshape(1, _HID),
                 batch.astype(jnp.int32).reshape(1, _N), cont,
                 box_idx.astype(jnp.int32).reshape(_NB, 1), box_table,
                 cp_w1, cp_b1.reshape(1, 64), cp_w2, cp_b2.reshape(1, 32),
                 mlp_w1, mlp_b1.reshape(1, 256), mlp_w2, mlp_b2.reshape(1, 128),
                 mlp_w3, mlp_b3.reshape(1, 1))
